# baseline (speedup 1.0000x reference)
import functools

import jax
import jax.numpy as jnp
from jax import lax
from jax.experimental import pallas as pl
from jax.experimental.pallas import tpu as pltpu
from jax.experimental.pallas import tpu_sc as plsc

N = 10000
E = 320000
D_IN = 128
H = 64
HH = H // 2
OUT = 2
G = 128

NTILES = 32
CHUNK = 256
TCH = 2560
E_PAD = TCH * CHUNK
CPT = TCH // 16
DCH = TCH // NTILES
N_PAD = 10112
RPT = N_PAD // 16
DEG_W = 16

NSLOT = 4
AHEAD = 2

BLK = 1000
NB = N // BLK


@functools.cache
def _sc_degree_call():
    mesh = plsc.VectorSubcoreMesh(core_axis_name="c", subcore_axis_name="s")
    return pl.kernel(
        _sc_degree,
        out_type=[jax.ShapeDtypeStruct((N_PAD, DEG_W), jnp.float32),
                  jax.ShapeDtypeStruct((N_PAD, DEG_W), jnp.float32)],
        mesh=mesh,
        scratch_types=[
            pltpu.VMEM((DCH, CHUNK), jnp.int32),
            pltpu.VMEM((CHUNK, DEG_W), jnp.float32),
            pltpu.VMEM_SHARED((N_PAD, DEG_W), jnp.float32),
        ],
        compiler_params=pltpu.CompilerParams(use_tc_tiling_on_sc=False),
    )


def _sc_degree(dst_hbm, z16_hbm, deg0_hbm, deg1_hbm, dst_v, ones_v, acc_sh):
    cid = lax.axis_index("c")
    sid = lax.axis_index("s")
    wid = sid * 2 + cid
    r0 = sid * RPT
    for r in range(CHUNK):
        ones_v[r] = jnp.ones((16,), jnp.float32)
    pltpu.sync_copy(z16_hbm, acc_sh.at[pl.ds(r0, RPT)])
    pltpu.sync_copy(dst_hbm.at[pl.ds(wid * DCH, DCH)], dst_v)
    plsc.subcore_barrier()

    @pl.loop(0, DCH)
    def _(j):
        pltpu.sync_copy(ones_v, acc_sh.at[dst_v.at[j]], add=True)

    plsc.subcore_barrier()

    @pl.when(cid == 0)
    def _():
        pltpu.sync_copy(acc_sh.at[pl.ds(r0, RPT)], deg0_hbm.at[pl.ds(r0, RPT)])

    @pl.when(cid == 1)
    def _():
        pltpu.sync_copy(acc_sh.at[pl.ds(r0, RPT)], deg1_hbm.at[pl.ds(r0, RPT)])


@functools.cache
def _sc_edge_call():
    mesh = plsc.VectorSubcoreMesh(core_axis_name="c", subcore_axis_name="s")
    return pl.kernel(
        _sc_edge,
        out_type=[jax.ShapeDtypeStruct((N_PAD, HH), jnp.float32),
                  jax.ShapeDtypeStruct((N_PAD, HH), jnp.float32)],
        mesh=mesh,
        scratch_types=[
            pltpu.VMEM((CPT // 2, CHUNK), jnp.int32),
            pltpu.VMEM((CPT // 2, CHUNK), jnp.int32),
            pltpu.VMEM((NSLOT, CHUNK, HH), jnp.float32),
            pltpu.VMEM_SHARED((N_PAD, HH), jnp.float32),
            pltpu.VMEM_SHARED((N_PAD, HH), jnp.float32),
        ] + [pltpu.SemaphoreType.DMA] * (2 * NSLOT),
        compiler_params=pltpu.CompilerParams(use_tc_tiling_on_sc=False),
    )


def _sc_edge(src_hbm, dst_hbm, ga_hbm, gb_hbm, z32_hbm, acca_hbm, accb_hbm,
             src_v, dst_v, rows_v, acc_sh, g_sh, *sems):
    cid = lax.axis_index("c")
    sid = lax.axis_index("s")
    r0 = sid * RPT
    base = sid * CPT
    pltpu.sync_copy(z32_hbm, acc_sh.at[pl.ds(r0, RPT)])

    @pl.when(cid == 0)
    def _():
        pltpu.sync_copy(ga_hbm.at[pl.ds(r0, RPT)], g_sh.at[pl.ds(r0, RPT)])

    @pl.when(cid == 1)
    def _():
        pltpu.sync_copy(gb_hbm.at[pl.ds(r0, RPT)], g_sh.at[pl.ds(r0, RPT)])

    plsc.subcore_barrier()

    sem_g = sems[:NSLOT]
    sem_s = sems[NSLOT:]
    HCPT = CPT // 2

    def wait_gather(c, b):
        pltpu.make_async_copy(
            g_sh.at[src_v.at[c]], rows_v.at[b], sem_g[b]).wait()

    def wait_scatter(c, b):
        pltpu.make_async_copy(
            rows_v.at[b], acc_sh.at[dst_v.at[c]], sem_s[b]).wait()

    def step(c, b, nxt_guard, wait_prev_scatter):
        wait_gather(c, b)
        pltpu.async_copy(rows_v.at[b], acc_sh.at[dst_v.at[c]], sem_s[b],
                         add=True)
        if nxt_guard:
            n = c + AHEAD
            bb = (b + AHEAD) % NSLOT
            if wait_prev_scatter:
                wait_scatter(n - NSLOT, bb)
            pltpu.async_copy(g_sh.at[src_v.at[n]], rows_v.at[bb], sem_g[bb])

    for phase in range(2):
        pltpu.sync_copy(src_hbm.at[pl.ds(base + phase * HCPT, HCPT)], src_v)
        pltpu.sync_copy(dst_hbm.at[pl.ds(base + phase * HCPT, HCPT)], dst_v)

        for c in range(AHEAD):
            pltpu.async_copy(g_sh.at[src_v.at[c]], rows_v.at[c], sem_g[c])

        for c in range(NSLOT):
            step(c, c % NSLOT, True, c + AHEAD >= NSLOT)

        @pl.loop(1, HCPT // NSLOT - 1)
        def _(j):
            c0 = j * NSLOT
            for b in range(NSLOT):
                step(c0 + b, b, True, True)

        for b in range(NSLOT):
            c = HCPT - NSLOT + b
            step(c, b, c + AHEAD < HCPT, True)
            wait_scatter(c, b)

    plsc.subcore_barrier()

    @pl.when(cid == 0)
    def _():
        pltpu.sync_copy(acc_sh.at[pl.ds(r0, RPT)], acca_hbm.at[pl.ds(r0, RPT)])

    @pl.when(cid == 1)
    def _():
        pltpu.sync_copy(acc_sh.at[pl.ds(r0, RPT)], accb_hbm.at[pl.ds(r0, RPT)])


def _tc_a1(x_ref, w1_ref, h_ref):
    h_ref[...] = jnp.dot(x_ref[...], w1_ref[...],
                         preferred_element_type=jnp.float32)


def _tc_a2(h_ref, d0_ref, d1_ref, ga_ref, gb_ref, dinv_ref):
    deg = 1.0 + d0_ref[:, 0:1] + d1_ref[:, 0:1]
    dinv = lax.rsqrt(jnp.maximum(deg, 1.0))
    dinvb = jnp.broadcast_to(dinv, (BLK, H))
    g = h_ref[...] * dinvb
    ga_ref[...] = g[:, :HH]
    gb_ref[...] = g[:, HH:]
    dinv_ref[...] = dinvb


def _tc_b(aa_ref, ab_ref, ga_ref, gb_ref, dinv_ref, b1_ref, w2_ref,
          g2a_ref, g2b_ref):
    dinvb = dinv_ref[...]
    acc = jnp.concatenate([aa_ref[...], ab_ref[...]], axis=1)
    g1 = jnp.concatenate([ga_ref[...], gb_ref[...]], axis=1)
    out1 = jnp.maximum(dinvb * (acc + g1) + b1_ref[...], 0.0)
    g2 = jnp.dot(out1, w2_ref[...], preferred_element_type=jnp.float32) * dinvb
    g2a_ref[...] = g2[:, :HH]
    g2b_ref[...] = g2[:, HH:]


def _tc_c(aa_ref, ab_ref, ga_ref, gb_ref, dinv_ref, b2_ref, batch_ref,
          wc_ref, bc_ref, out_ref, psum, pcnt):
    i = pl.program_id(0)

    @pl.when(i == 0)
    def _():
        psum[...] = jnp.zeros_like(psum)
        pcnt[...] = jnp.zeros_like(pcnt)

    dinvb = dinv_ref[...]
    acc = jnp.concatenate([aa_ref[...], ab_ref[...]], axis=1)
    g2 = jnp.concatenate([ga_ref[...], gb_ref[...]], axis=1)
    out2 = jnp.maximum(dinvb * (acc + g2) + b2_ref[...], 0.0)
    ids = batch_ref[0]
    iota = lax.broadcasted_iota(jnp.int32, (G, BLK), 0)
    onehot = (iota == ids).astype(jnp.float32)
    psum[...] += jnp.dot(onehot, out2, preferred_element_type=jnp.float32)
    pcnt[...] += jnp.dot(onehot, jnp.ones((BLK, 8), jnp.float32),
                         preferred_element_type=jnp.float32)

    @pl.when(i == NB - 1)
    def _():
        pooled = psum[...] / jnp.maximum(pcnt[:, 0:1], 1.0)
        out_ref[...] = jnp.dot(
            pooled, wc_ref[...], preferred_element_type=jnp.float32) + bc_ref[...]


def _row_spec(width):
    return pl.BlockSpec((BLK, width), lambda i: (i, 0))


def _full_spec(shape):
    return pl.BlockSpec(shape, lambda i: tuple(0 for _ in shape))


def _half_shapes():
    return [jax.ShapeDtypeStruct((N_PAD, HH), jnp.float32),
            jax.ShapeDtypeStruct((N_PAD, HH), jnp.float32)]


_tc_a1_call = pl.pallas_call(
    _tc_a1,
    grid=(NB,),
    in_specs=[_row_spec(D_IN), _full_spec((D_IN, H))],
    out_specs=_row_spec(H),
    out_shape=jax.ShapeDtypeStruct((N, H), jnp.float32),
)

_tc_a2_call = pl.pallas_call(
    _tc_a2,
    grid=(NB,),
    in_specs=[_row_spec(H), _row_spec(DEG_W), _row_spec(DEG_W)],
    out_specs=[_row_spec(HH), _row_spec(HH), _row_spec(H)],
    out_shape=_half_shapes() + [jax.ShapeDtypeStruct((N, H), jnp.float32)],
)

_tc_b_call = pl.pallas_call(
    _tc_b,
    grid=(NB,),
    in_specs=[_row_spec(HH), _row_spec(HH), _row_spec(HH), _row_spec(HH),
              _row_spec(H), _full_spec((1, H)), _full_spec((H, H))],
    out_specs=[_row_spec(HH), _row_spec(HH)],
    out_shape=_half_shapes(),
)

_tc_c_call = pl.pallas_call(
    _tc_c,
    grid=(NB,),
    in_specs=[_row_spec(HH), _row_spec(HH), _row_spec(HH), _row_spec(HH),
              _row_spec(H), _full_spec((1, H)),
              pl.BlockSpec((1, 1, BLK), lambda i: (i, 0, 0)),
              _full_spec((H, OUT)), _full_spec((1, OUT))],
    out_specs=_full_spec((G, OUT)),
    out_shape=jax.ShapeDtypeStruct((G, OUT), jnp.float32),
    scratch_shapes=[pltpu.VMEM((G, H), jnp.float32),
                    pltpu.VMEM((G, 8), jnp.float32)],
)


@jax.jit
def kernel(x, edge_index, batch, W1, b1, W2, b2, Wc, bc):
    src = edge_index[0]
    dst = edge_index[1]
    pad = E_PAD - E
    src2 = jnp.concatenate([src, jnp.zeros((pad,), jnp.int32)]).reshape(
        TCH, CHUNK)
    dst2 = jnp.concatenate([dst, jnp.full((pad,), N, jnp.int32)]).reshape(
        TCH, CHUNK)
    z16 = jnp.zeros((RPT, DEG_W), jnp.float32)
    z32 = jnp.zeros((RPT, HH), jnp.float32)
    batch3 = batch.reshape(NB, 1, BLK)

    deg0, deg1 = _sc_degree_call()(dst2, z16)
    h1 = _tc_a1_call(x, W1)
    g1a, g1b, dinvb = _tc_a2_call(h1, deg0, deg1)
    a0, a1 = _sc_edge_call()(src2, dst2, g1a, g1b, z32)
    g2a, g2b = _tc_b_call(a0, a1, g1a, g1b, dinvb, b1.reshape(1, H), W2)
    c0, c1 = _sc_edge_call()(src2, dst2, g2a, g2b, z32)
    return _tc_c_call(c0, c1, g2a, g2b, dinvb, b2.reshape(1, H), batch3,
                      Wc, bc.reshape(1, OUT))


# speedup vs baseline: 4.3035x; 4.3035x over previous
import functools

import jax
import jax.numpy as jnp
from jax import lax
from jax.experimental import pallas as pl
from jax.experimental.pallas import tpu as pltpu
from jax.experimental.pallas import tpu_sc as plsc

N = 10000
E = 320000
D_IN = 128
H = 64
HH = H // 2
OUT = 2
G = 128

NTILES = 32
CHUNK = 128
TCH = 2560
E_PAD = TCH * CHUNK
CPT = TCH // 16
DCH = TCH // NTILES
N_PAD = 10112
RPT = N_PAD // 16
DEG_W = 16

NSLOT = 8
AHEAD = 4

BLK = 1000
NB = N // BLK


@functools.cache
def _sc_degree_call():
    mesh = plsc.VectorSubcoreMesh(core_axis_name="c", subcore_axis_name="s")
    return pl.kernel(
        _sc_degree,
        out_type=[jax.ShapeDtypeStruct((N_PAD, DEG_W), jnp.float32),
                  jax.ShapeDtypeStruct((N_PAD, DEG_W), jnp.float32)],
        mesh=mesh,
        scratch_types=[
            pltpu.VMEM((DCH, CHUNK), jnp.int32),
            pltpu.VMEM((CHUNK, DEG_W), jnp.float32),
            pltpu.VMEM_SHARED((N_PAD, DEG_W), jnp.float32),
        ],
        compiler_params=pltpu.CompilerParams(use_tc_tiling_on_sc=False),
    )


def _sc_degree(dst_hbm, z16_hbm, deg0_hbm, deg1_hbm, dst_v, ones_v, acc_sh):
    cid = lax.axis_index("c")
    sid = lax.axis_index("s")
    wid = sid * 2 + cid
    r0 = sid * RPT
    for r in range(CHUNK):
        ones_v[r] = jnp.ones((16,), jnp.float32)
    pltpu.sync_copy(z16_hbm, acc_sh.at[pl.ds(r0, RPT)])
    pltpu.sync_copy(dst_hbm.at[pl.ds(wid * DCH, DCH)], dst_v)
    plsc.subcore_barrier()

    @pl.loop(0, DCH)
    def _(j):
        pltpu.sync_copy(ones_v, acc_sh.at[dst_v.at[j]], add=True)

    plsc.subcore_barrier()

    @pl.when(cid == 0)
    def _():
        pltpu.sync_copy(acc_sh.at[pl.ds(r0, RPT)], deg0_hbm.at[pl.ds(r0, RPT)])

    @pl.when(cid == 1)
    def _():
        pltpu.sync_copy(acc_sh.at[pl.ds(r0, RPT)], deg1_hbm.at[pl.ds(r0, RPT)])


@functools.cache
def _sc_edge_call():
    mesh = plsc.VectorSubcoreMesh(core_axis_name="c", subcore_axis_name="s")
    return pl.kernel(
        _sc_edge,
        out_type=[jax.ShapeDtypeStruct((N_PAD, HH), jnp.float32),
                  jax.ShapeDtypeStruct((N_PAD, HH), jnp.float32)],
        mesh=mesh,
        scratch_types=[
            pltpu.VMEM((CPT // 2, CHUNK), jnp.int32),
            pltpu.VMEM((CPT // 2, CHUNK), jnp.int32),
            pltpu.VMEM((NSLOT, CHUNK, HH), jnp.float32),
            pltpu.VMEM_SHARED((N_PAD, HH), jnp.float32),
            pltpu.VMEM_SHARED((N_PAD, HH), jnp.float32),
        ] + [pltpu.SemaphoreType.DMA] * (2 * NSLOT),
        compiler_params=pltpu.CompilerParams(use_tc_tiling_on_sc=False),
    )


def _sc_edge(src_hbm, dst_hbm, ga_hbm, gb_hbm, z32_hbm, acca_hbm, accb_hbm,
             src_v, dst_v, rows_v, acc_sh, g_sh, *sems):
    cid = lax.axis_index("c")
    sid = lax.axis_index("s")
    r0 = sid * RPT
    base = sid * CPT
    pltpu.sync_copy(z32_hbm, acc_sh.at[pl.ds(r0, RPT)])

    @pl.when(cid == 0)
    def _():
        pltpu.sync_copy(ga_hbm.at[pl.ds(r0, RPT)], g_sh.at[pl.ds(r0, RPT)])

    @pl.when(cid == 1)
    def _():
        pltpu.sync_copy(gb_hbm.at[pl.ds(r0, RPT)], g_sh.at[pl.ds(r0, RPT)])

    plsc.subcore_barrier()

    sem_g = sems[:NSLOT]
    sem_s = sems[NSLOT:]
    HCPT = CPT // 2

    def wait_gather(c, b):
        pltpu.make_async_copy(
            g_sh.at[src_v.at[c]], rows_v.at[b], sem_g[b]).wait()

    def wait_scatter(c, b):
        pltpu.make_async_copy(
            rows_v.at[b], acc_sh.at[dst_v.at[c]], sem_s[b]).wait()

    def step(c, b, nxt_guard, wait_prev_scatter):
        wait_gather(c, b)
        pltpu.async_copy(rows_v.at[b], acc_sh.at[dst_v.at[c]], sem_s[b],
                         add=True)
        if nxt_guard:
            n = c + AHEAD
            bb = (b + AHEAD) % NSLOT
            if wait_prev_scatter:
                wait_scatter(n - NSLOT, bb)
            pltpu.async_copy(g_sh.at[src_v.at[n]], rows_v.at[bb], sem_g[bb])

    for phase in range(2):
        pltpu.sync_copy(src_hbm.at[pl.ds(base + phase * HCPT, HCPT)], src_v)
        pltpu.sync_copy(dst_hbm.at[pl.ds(base + phase * HCPT, HCPT)], dst_v)

        for c in range(AHEAD):
            pltpu.async_copy(g_sh.at[src_v.at[c]], rows_v.at[c], sem_g[c])

        for c in range(NSLOT):
            step(c, c % NSLOT, True, c + AHEAD >= NSLOT)

        @pl.loop(1, HCPT // NSLOT - 1)
        def _(j):
            c0 = j * NSLOT
            for b in range(NSLOT):
                step(c0 + b, b, True, True)

        for b in range(NSLOT):
            c = HCPT - NSLOT + b
            step(c, b, c + AHEAD < HCPT, True)
            wait_scatter(c, b)

    plsc.subcore_barrier()

    @pl.when(cid == 0)
    def _():
        pltpu.sync_copy(acc_sh.at[pl.ds(r0, RPT)], acca_hbm.at[pl.ds(r0, RPT)])

    @pl.when(cid == 1)
    def _():
        pltpu.sync_copy(acc_sh.at[pl.ds(r0, RPT)], accb_hbm.at[pl.ds(r0, RPT)])


def _tc_a1(x_ref, w1_ref, h_ref):
    h_ref[...] = jnp.dot(x_ref[...], w1_ref[...],
                         preferred_element_type=jnp.float32)


def _tc_a2(h_ref, d0_ref, d1_ref, ga_ref, gb_ref, dinv_ref):
    deg = 1.0 + d0_ref[:, 0:1] + d1_ref[:, 0:1]
    dinv = lax.rsqrt(jnp.maximum(deg, 1.0))
    dinvb = jnp.broadcast_to(dinv, (BLK, H))
    g = h_ref[...] * dinvb
    ga_ref[...] = g[:, :HH]
    gb_ref[...] = g[:, HH:]
    dinv_ref[...] = dinvb


def _tc_b(aa_ref, ab_ref, ga_ref, gb_ref, dinv_ref, b1_ref, w2_ref,
          g2a_ref, g2b_ref):
    dinvb = dinv_ref[...]
    acc = jnp.concatenate([aa_ref[...], ab_ref[...]], axis=1)
    g1 = jnp.concatenate([ga_ref[...], gb_ref[...]], axis=1)
    out1 = jnp.maximum(dinvb * (acc + g1) + b1_ref[...], 0.0)
    g2 = jnp.dot(out1, w2_ref[...], preferred_element_type=jnp.float32) * dinvb
    g2a_ref[...] = g2[:, :HH]
    g2b_ref[...] = g2[:, HH:]


def _tc_c(aa_ref, ab_ref, ga_ref, gb_ref, dinv_ref, b2_ref, batch_ref,
          wc_ref, bc_ref, out_ref, psum, pcnt):
    i = pl.program_id(0)

    @pl.when(i == 0)
    def _():
        psum[...] = jnp.zeros_like(psum)
        pcnt[...] = jnp.zeros_like(pcnt)

    dinvb = dinv_ref[...]
    acc = jnp.concatenate([aa_ref[...], ab_ref[...]], axis=1)
    g2 = jnp.concatenate([ga_ref[...], gb_ref[...]], axis=1)
    out2 = jnp.maximum(dinvb * (acc + g2) + b2_ref[...], 0.0)
    ids = batch_ref[0]
    iota = lax.broadcasted_iota(jnp.int32, (G, BLK), 0)
    onehot = (iota == ids).astype(jnp.float32)
    psum[...] += jnp.dot(onehot, out2, preferred_element_type=jnp.float32)
    pcnt[...] += jnp.dot(onehot, jnp.ones((BLK, 8), jnp.float32),
                         preferred_element_type=jnp.float32)

    @pl.when(i == NB - 1)
    def _():
        pooled = psum[...] / jnp.maximum(pcnt[:, 0:1], 1.0)
        out_ref[...] = jnp.dot(
            pooled, wc_ref[...], preferred_element_type=jnp.float32) + bc_ref[...]


def _row_spec(width):
    return pl.BlockSpec((BLK, width), lambda i: (i, 0))


def _full_spec(shape):
    return pl.BlockSpec(shape, lambda i: tuple(0 for _ in shape))


def _half_shapes():
    return [jax.ShapeDtypeStruct((N_PAD, HH), jnp.float32),
            jax.ShapeDtypeStruct((N_PAD, HH), jnp.float32)]


_tc_a1_call = pl.pallas_call(
    _tc_a1,
    grid=(NB,),
    in_specs=[_row_spec(D_IN), _full_spec((D_IN, H))],
    out_specs=_row_spec(H),
    out_shape=jax.ShapeDtypeStruct((N, H), jnp.float32),
)

_tc_a2_call = pl.pallas_call(
    _tc_a2,
    grid=(NB,),
    in_specs=[_row_spec(H), _row_spec(DEG_W), _row_spec(DEG_W)],
    out_specs=[_row_spec(HH), _row_spec(HH), _row_spec(H)],
    out_shape=_half_shapes() + [jax.ShapeDtypeStruct((N, H), jnp.float32)],
)

_tc_b_call = pl.pallas_call(
    _tc_b,
    grid=(NB,),
    in_specs=[_row_spec(HH), _row_spec(HH), _row_spec(HH), _row_spec(HH),
              _row_spec(H), _full_spec((1, H)), _full_spec((H, H))],
    out_specs=[_row_spec(HH), _row_spec(HH)],
    out_shape=_half_shapes(),
)

_tc_c_call = pl.pallas_call(
    _tc_c,
    grid=(NB,),
    in_specs=[_row_spec(HH), _row_spec(HH), _row_spec(HH), _row_spec(HH),
              _row_spec(H), _full_spec((1, H)),
              pl.BlockSpec((1, 1, BLK), lambda i: (i, 0, 0)),
              _full_spec((H, OUT)), _full_spec((1, OUT))],
    out_specs=_full_spec((G, OUT)),
    out_shape=jax.ShapeDtypeStruct((G, OUT), jnp.float32),
    scratch_shapes=[pltpu.VMEM((G, H), jnp.float32),
                    pltpu.VMEM((G, 8), jnp.float32)],
)


@jax.jit
def kernel(x, edge_index, batch, W1, b1, W2, b2, Wc, bc):
    src = edge_index[0]
    dst = edge_index[1]
    pad = E_PAD - E
    src2 = jnp.concatenate([src, jnp.zeros((pad,), jnp.int32)]).reshape(
        TCH, CHUNK)
    dst2 = jnp.concatenate([dst, jnp.full((pad,), N, jnp.int32)]).reshape(
        TCH, CHUNK)
    z16 = jnp.zeros((RPT, DEG_W), jnp.float32)
    z32 = jnp.zeros((RPT, HH), jnp.float32)
    batch3 = batch.reshape(NB, 1, BLK)

    deg0, deg1 = _sc_degree_call()(dst2, z16)
    h1 = _tc_a1_call(x, W1)
    g1a, g1b, dinvb = _tc_a2_call(h1, deg0, deg1)
    a0, a1 = _sc_edge_call()(src2, dst2, g1a, g1b, z32)
    g2a, g2b = _tc_b_call(a0, a1, g1a, g1b, dinvb, b1.reshape(1, H), W2)
    c0, c1 = _sc_edge_call()(src2, dst2, g2a, g2b, z32)
    return _tc_c_call(c0, c1, g2a, g2b, dinvb, b2.reshape(1, H), batch3,
                      Wc, bc.reshape(1, OUT))


# speedup vs baseline: 5.6180x; 1.3054x over previous
import functools

import jax
import jax.numpy as jnp
from jax import lax
from jax.experimental import pallas as pl
from jax.experimental.pallas import tpu as pltpu
from jax.experimental.pallas import tpu_sc as plsc

N = 10000
E = 320000
D_IN = 128
H = 64
HH = H // 2
OUT = 2
G = 128

NTILES = 32
CHUNK = 128
TCH = 2560
E_PAD = TCH * CHUNK
CPT = TCH // 16
DCH = TCH // NTILES
N_PAD = 10112
RPT = N_PAD // 16
DEG_W = 16

NSLOT = 8
AHEAD = 4

BLK = 1000
NB = N // BLK


@functools.cache
def _sc_degree_call():
    mesh = plsc.VectorSubcoreMesh(core_axis_name="c", subcore_axis_name="s")
    return pl.kernel(
        _sc_degree,
        out_type=[jax.ShapeDtypeStruct((N_PAD, DEG_W), jnp.float32),
                  jax.ShapeDtypeStruct((N_PAD, DEG_W), jnp.float32)],
        mesh=mesh,
        scratch_types=[
            pltpu.VMEM((DCH, CHUNK), jnp.int32),
            pltpu.VMEM((CHUNK, DEG_W), jnp.float32),
            pltpu.VMEM_SHARED((N_PAD, DEG_W), jnp.float32),
        ],
        compiler_params=pltpu.CompilerParams(use_tc_tiling_on_sc=False),
    )


def _sc_degree(dst_hbm, z16_hbm, deg0_hbm, deg1_hbm, dst_v, ones_v, acc_sh):
    cid = lax.axis_index("c")
    sid = lax.axis_index("s")
    wid = sid * 2 + cid
    r0 = sid * RPT
    for r in range(CHUNK):
        ones_v[r] = jnp.ones((16,), jnp.float32)
    pltpu.sync_copy(z16_hbm, acc_sh.at[pl.ds(r0, RPT)])
    pltpu.sync_copy(dst_hbm.at[pl.ds(wid * DCH, DCH)], dst_v)
    plsc.subcore_barrier()

    @pl.loop(0, DCH)
    def _(j):
        pltpu.sync_copy(ones_v, acc_sh.at[dst_v.at[j]], add=True)

    plsc.subcore_barrier()

    @pl.when(cid == 0)
    def _():
        pltpu.sync_copy(acc_sh.at[pl.ds(r0, RPT)], deg0_hbm.at[pl.ds(r0, RPT)])

    @pl.when(cid == 1)
    def _():
        pltpu.sync_copy(acc_sh.at[pl.ds(r0, RPT)], deg1_hbm.at[pl.ds(r0, RPT)])


@functools.cache
def _sc_edge_call():
    mesh = plsc.VectorSubcoreMesh(core_axis_name="c", subcore_axis_name="s")
    return pl.kernel(
        _sc_edge,
        out_type=[jax.ShapeDtypeStruct((N_PAD, HH), jnp.bfloat16),
                  jax.ShapeDtypeStruct((N_PAD, HH), jnp.bfloat16)],
        mesh=mesh,
        scratch_types=[
            pltpu.VMEM((CPT, CHUNK), jnp.int32),
            pltpu.VMEM((CPT, CHUNK), jnp.int32),
            pltpu.VMEM((NSLOT, CHUNK, HH), jnp.bfloat16),
            pltpu.VMEM_SHARED((N_PAD, HH), jnp.bfloat16),
            pltpu.VMEM_SHARED((N_PAD, HH), jnp.bfloat16),
        ] + [pltpu.SemaphoreType.DMA] * (2 * NSLOT),
        compiler_params=pltpu.CompilerParams(use_tc_tiling_on_sc=False),
    )


def _sc_edge(src_hbm, dst_hbm, ga_hbm, gb_hbm, z32_hbm, acca_hbm, accb_hbm,
             src_v, dst_v, rows_v, acc_sh, g_sh, *sems):
    cid = lax.axis_index("c")
    sid = lax.axis_index("s")
    r0 = sid * RPT
    base = sid * CPT
    pltpu.sync_copy(z32_hbm, acc_sh.at[pl.ds(r0, RPT)])

    @pl.when(cid == 0)
    def _():
        pltpu.sync_copy(ga_hbm.at[pl.ds(r0, RPT)], g_sh.at[pl.ds(r0, RPT)])

    @pl.when(cid == 1)
    def _():
        pltpu.sync_copy(gb_hbm.at[pl.ds(r0, RPT)], g_sh.at[pl.ds(r0, RPT)])

    pltpu.sync_copy(src_hbm.at[pl.ds(base, CPT)], src_v)
    pltpu.sync_copy(dst_hbm.at[pl.ds(base, CPT)], dst_v)
    plsc.subcore_barrier()

    sem_g = sems[:NSLOT]
    sem_s = sems[NSLOT:]

    def wait_gather(c, b):
        pltpu.make_async_copy(
            g_sh.at[src_v.at[c]], rows_v.at[b], sem_g[b]).wait()

    def wait_scatter(c, b):
        pltpu.make_async_copy(
            rows_v.at[b], acc_sh.at[dst_v.at[c]], sem_s[b]).wait()

    def step(c, b, nxt_guard, wait_prev_scatter):
        wait_gather(c, b)
        pltpu.async_copy(rows_v.at[b], acc_sh.at[dst_v.at[c]], sem_s[b],
                         add=True)
        if nxt_guard:
            n = c + AHEAD
            bb = (b + AHEAD) % NSLOT
            if wait_prev_scatter:
                wait_scatter(n - NSLOT, bb)
            pltpu.async_copy(g_sh.at[src_v.at[n]], rows_v.at[bb], sem_g[bb])

    for c in range(AHEAD):
        pltpu.async_copy(g_sh.at[src_v.at[c]], rows_v.at[c], sem_g[c])

    for c in range(NSLOT):
        step(c, c % NSLOT, True, c + AHEAD >= NSLOT)

    @pl.loop(1, CPT // NSLOT - 1)
    def _(j):
        c0 = j * NSLOT
        for b in range(NSLOT):
            step(c0 + b, b, True, True)

    for b in range(NSLOT):
        c = CPT - NSLOT + b
        step(c, b, c + AHEAD < CPT, True)
        wait_scatter(c, b)

    plsc.subcore_barrier()

    @pl.when(cid == 0)
    def _():
        pltpu.sync_copy(acc_sh.at[pl.ds(r0, RPT)], acca_hbm.at[pl.ds(r0, RPT)])

    @pl.when(cid == 1)
    def _():
        pltpu.sync_copy(acc_sh.at[pl.ds(r0, RPT)], accb_hbm.at[pl.ds(r0, RPT)])


def _tc_a1(x_ref, w1_ref, h_ref):
    h_ref[...] = jnp.dot(x_ref[...], w1_ref[...],
                         preferred_element_type=jnp.float32)


def _tc_a2(h_ref, d0_ref, d1_ref, ga_ref, gb_ref, dinv_ref):
    deg = 1.0 + d0_ref[:, 0:1] + d1_ref[:, 0:1]
    dinv = lax.rsqrt(jnp.maximum(deg, 1.0))
    dinvb = jnp.broadcast_to(dinv, (BLK, H))
    g = h_ref[...] * dinvb
    ga_ref[...] = g[:, :HH].astype(jnp.bfloat16)
    gb_ref[...] = g[:, HH:].astype(jnp.bfloat16)
    dinv_ref[...] = dinvb


def _tc_b(aa_ref, ab_ref, ga_ref, gb_ref, dinv_ref, b1_ref, w2_ref,
          g2a_ref, g2b_ref):
    dinvb = dinv_ref[...]
    acc = jnp.concatenate([aa_ref[...], ab_ref[...]], axis=1).astype(
        jnp.float32)
    g1 = jnp.concatenate([ga_ref[...], gb_ref[...]], axis=1).astype(
        jnp.float32)
    out1 = jnp.maximum(dinvb * (acc + g1) + b1_ref[...], 0.0)
    g2 = jnp.dot(out1, w2_ref[...], preferred_element_type=jnp.float32) * dinvb
    g2a_ref[...] = g2[:, :HH].astype(jnp.bfloat16)
    g2b_ref[...] = g2[:, HH:].astype(jnp.bfloat16)


def _tc_c(aa_ref, ab_ref, ga_ref, gb_ref, dinv_ref, b2_ref, batch_ref,
          wc_ref, bc_ref, out_ref, psum, pcnt):
    i = pl.program_id(0)

    @pl.when(i == 0)
    def _():
        psum[...] = jnp.zeros_like(psum)
        pcnt[...] = jnp.zeros_like(pcnt)

    dinvb = dinv_ref[...]
    acc = jnp.concatenate([aa_ref[...], ab_ref[...]], axis=1).astype(
        jnp.float32)
    g2 = jnp.concatenate([ga_ref[...], gb_ref[...]], axis=1).astype(
        jnp.float32)
    out2 = jnp.maximum(dinvb * (acc + g2) + b2_ref[...], 0.0)
    ids = batch_ref[0]
    iota = lax.broadcasted_iota(jnp.int32, (G, BLK), 0)
    onehot = (iota == ids).astype(jnp.float32)
    psum[...] += jnp.dot(onehot, out2, preferred_element_type=jnp.float32)
    pcnt[...] += jnp.dot(onehot, jnp.ones((BLK, 8), jnp.float32),
                         preferred_element_type=jnp.float32)

    @pl.when(i == NB - 1)
    def _():
        pooled = psum[...] / jnp.maximum(pcnt[:, 0:1], 1.0)
        out_ref[...] = jnp.dot(
            pooled, wc_ref[...], preferred_element_type=jnp.float32) + bc_ref[...]


def _row_spec(width):
    return pl.BlockSpec((BLK, width), lambda i: (i, 0))


def _full_spec(shape):
    return pl.BlockSpec(shape, lambda i: tuple(0 for _ in shape))


def _half_shapes():
    return [jax.ShapeDtypeStruct((N_PAD, HH), jnp.bfloat16),
            jax.ShapeDtypeStruct((N_PAD, HH), jnp.bfloat16)]


_tc_a1_call = pl.pallas_call(
    _tc_a1,
    grid=(NB,),
    in_specs=[_row_spec(D_IN), _full_spec((D_IN, H))],
    out_specs=_row_spec(H),
    out_shape=jax.ShapeDtypeStruct((N, H), jnp.float32),
)

_tc_a2_call = pl.pallas_call(
    _tc_a2,
    grid=(NB,),
    in_specs=[_row_spec(H), _row_spec(DEG_W), _row_spec(DEG_W)],
    out_specs=[_row_spec(HH), _row_spec(HH), _row_spec(H)],
    out_shape=_half_shapes() + [jax.ShapeDtypeStruct((N, H), jnp.float32)],
)

_tc_b_call = pl.pallas_call(
    _tc_b,
    grid=(NB,),
    in_specs=[_row_spec(HH), _row_spec(HH), _row_spec(HH), _row_spec(HH),
              _row_spec(H), _full_spec((1, H)), _full_spec((H, H))],
    out_specs=[_row_spec(HH), _row_spec(HH)],
    out_shape=_half_shapes(),
)

_tc_c_call = pl.pallas_call(
    _tc_c,
    grid=(NB,),
    in_specs=[_row_spec(HH), _row_spec(HH), _row_spec(HH), _row_spec(HH),
              _row_spec(H), _full_spec((1, H)),
              pl.BlockSpec((1, 1, BLK), lambda i: (i, 0, 0)),
              _full_spec((H, OUT)), _full_spec((1, OUT))],
    out_specs=_full_spec((G, OUT)),
    out_shape=jax.ShapeDtypeStruct((G, OUT), jnp.float32),
    scratch_shapes=[pltpu.VMEM((G, H), jnp.float32),
                    pltpu.VMEM((G, 8), jnp.float32)],
)


@jax.jit
def kernel(x, edge_index, batch, W1, b1, W2, b2, Wc, bc):
    src = edge_index[0]
    dst = edge_index[1]
    pad = E_PAD - E
    src2 = jnp.concatenate([src, jnp.zeros((pad,), jnp.int32)]).reshape(
        TCH, CHUNK)
    dst2 = jnp.concatenate([dst, jnp.full((pad,), N, jnp.int32)]).reshape(
        TCH, CHUNK)
    z16 = jnp.zeros((RPT, DEG_W), jnp.float32)
    z32 = jnp.zeros((RPT, HH), jnp.bfloat16)
    batch3 = batch.reshape(NB, 1, BLK)

    deg0, deg1 = _sc_degree_call()(dst2, z16)
    h1 = _tc_a1_call(x, W1)
    g1a, g1b, dinvb = _tc_a2_call(h1, deg0, deg1)
    a0, a1 = _sc_edge_call()(src2, dst2, g1a, g1b, z32)
    g2a, g2b = _tc_b_call(a0, a1, g1a, g1b, dinvb, b1.reshape(1, H), W2)
    c0, c1 = _sc_edge_call()(src2, dst2, g2a, g2b, z32)
    return _tc_c_call(c0, c1, g2a, g2b, dinvb, b2.reshape(1, H), batch3,
                      Wc, bc.reshape(1, OUT))


# speedup vs baseline: 5.7674x; 1.0266x over previous
import functools

import jax
import jax.numpy as jnp
from jax import lax
from jax.experimental import pallas as pl
from jax.experimental.pallas import tpu as pltpu
from jax.experimental.pallas import tpu_sc as plsc

N = 10000
E = 320000
D_IN = 128
H = 64
HH = H // 2
OUT = 2
G = 128

NTILES = 32
CHUNK = 128
TCH = 2560
E_PAD = TCH * CHUNK
CPT = TCH // 16
DCH = TCH // NTILES
N_PAD = 10112
RPT = N_PAD // 16
DEG_W = 16

NSLOT = 8
AHEAD = 4

BLK = 2000
NB = N // BLK


@functools.cache
def _sc_degree_call():
    mesh = plsc.VectorSubcoreMesh(core_axis_name="c", subcore_axis_name="s")
    return pl.kernel(
        _sc_degree,
        out_type=[jax.ShapeDtypeStruct((N_PAD, DEG_W), jnp.float32),
                  jax.ShapeDtypeStruct((N_PAD, DEG_W), jnp.float32)],
        mesh=mesh,
        scratch_types=[
            pltpu.VMEM((DCH, CHUNK), jnp.int32),
            pltpu.VMEM((CHUNK, DEG_W), jnp.float32),
            pltpu.VMEM_SHARED((N_PAD, DEG_W), jnp.float32),
        ],
        compiler_params=pltpu.CompilerParams(use_tc_tiling_on_sc=False),
    )


def _sc_degree(dst_hbm, z16_hbm, deg0_hbm, deg1_hbm, dst_v, ones_v, acc_sh):
    cid = lax.axis_index("c")
    sid = lax.axis_index("s")
    wid = sid * 2 + cid
    r0 = sid * RPT
    for r in range(CHUNK):
        ones_v[r] = jnp.ones((16,), jnp.float32)
    pltpu.sync_copy(z16_hbm, acc_sh.at[pl.ds(r0, RPT)])
    pltpu.sync_copy(dst_hbm.at[pl.ds(wid * DCH, DCH)], dst_v)
    plsc.subcore_barrier()

    @pl.loop(0, DCH)
    def _(j):
        pltpu.sync_copy(ones_v, acc_sh.at[dst_v.at[j]], add=True)

    plsc.subcore_barrier()

    @pl.when(cid == 0)
    def _():
        pltpu.sync_copy(acc_sh.at[pl.ds(r0, RPT)], deg0_hbm.at[pl.ds(r0, RPT)])

    @pl.when(cid == 1)
    def _():
        pltpu.sync_copy(acc_sh.at[pl.ds(r0, RPT)], deg1_hbm.at[pl.ds(r0, RPT)])


@functools.cache
def _sc_edge_call():
    mesh = plsc.VectorSubcoreMesh(core_axis_name="c", subcore_axis_name="s")
    return pl.kernel(
        _sc_edge,
        out_type=[jax.ShapeDtypeStruct((N_PAD, HH), jnp.bfloat16),
                  jax.ShapeDtypeStruct((N_PAD, HH), jnp.bfloat16)],
        mesh=mesh,
        scratch_types=[
            pltpu.VMEM((CPT, CHUNK), jnp.int32),
            pltpu.VMEM((CPT, CHUNK), jnp.int32),
            pltpu.VMEM((NSLOT, CHUNK, HH), jnp.bfloat16),
            pltpu.VMEM_SHARED((N_PAD, HH), jnp.bfloat16),
            pltpu.VMEM_SHARED((N_PAD, HH), jnp.bfloat16),
        ] + [pltpu.SemaphoreType.DMA] * (2 * NSLOT),
        compiler_params=pltpu.CompilerParams(use_tc_tiling_on_sc=False),
    )


def _sc_edge(src_hbm, dst_hbm, ga_hbm, gb_hbm, z32_hbm, acca_hbm, accb_hbm,
             src_v, dst_v, rows_v, acc_sh, g_sh, *sems):
    cid = lax.axis_index("c")
    sid = lax.axis_index("s")
    r0 = sid * RPT
    base = sid * CPT
    pltpu.sync_copy(z32_hbm, acc_sh.at[pl.ds(r0, RPT)])

    @pl.when(cid == 0)
    def _():
        pltpu.sync_copy(ga_hbm.at[pl.ds(r0, RPT)], g_sh.at[pl.ds(r0, RPT)])

    @pl.when(cid == 1)
    def _():
        pltpu.sync_copy(gb_hbm.at[pl.ds(r0, RPT)], g_sh.at[pl.ds(r0, RPT)])

    pltpu.sync_copy(src_hbm.at[pl.ds(base, CPT)], src_v)
    pltpu.sync_copy(dst_hbm.at[pl.ds(base, CPT)], dst_v)
    plsc.subcore_barrier()

    sem_g = sems[:NSLOT]
    sem_s = sems[NSLOT:]

    def wait_gather(c, b):
        pltpu.make_async_copy(
            g_sh.at[src_v.at[c]], rows_v.at[b], sem_g[b]).wait()

    def wait_scatter(c, b):
        pltpu.make_async_copy(
            rows_v.at[b], acc_sh.at[dst_v.at[c]], sem_s[b]).wait()

    def step(c, b, nxt_guard, wait_prev_scatter):
        wait_gather(c, b)
        pltpu.async_copy(rows_v.at[b], acc_sh.at[dst_v.at[c]], sem_s[b],
                         add=True)
        if nxt_guard:
            n = c + AHEAD
            bb = (b + AHEAD) % NSLOT
            if wait_prev_scatter:
                wait_scatter(n - NSLOT, bb)
            pltpu.async_copy(g_sh.at[src_v.at[n]], rows_v.at[bb], sem_g[bb])

    for c in range(AHEAD):
        pltpu.async_copy(g_sh.at[src_v.at[c]], rows_v.at[c], sem_g[c])

    for c in range(NSLOT):
        step(c, c % NSLOT, True, c + AHEAD >= NSLOT)

    @pl.loop(1, CPT // NSLOT - 1)
    def _(j):
        c0 = j * NSLOT
        for b in range(NSLOT):
            step(c0 + b, b, True, True)

    for b in range(NSLOT):
        c = CPT - NSLOT + b
        step(c, b, c + AHEAD < CPT, True)
        wait_scatter(c, b)

    plsc.subcore_barrier()

    @pl.when(cid == 0)
    def _():
        pltpu.sync_copy(acc_sh.at[pl.ds(r0, RPT)], acca_hbm.at[pl.ds(r0, RPT)])

    @pl.when(cid == 1)
    def _():
        pltpu.sync_copy(acc_sh.at[pl.ds(r0, RPT)], accb_hbm.at[pl.ds(r0, RPT)])


def _tc_a1(x_ref, w1_ref, h_ref):
    h_ref[...] = jnp.dot(x_ref[...], w1_ref[...],
                         preferred_element_type=jnp.float32)


def _tc_a2(h_ref, d0_ref, d1_ref, ga_ref, gb_ref, dinv_ref):
    deg = 1.0 + d0_ref[:, 0:1] + d1_ref[:, 0:1]
    dinv = lax.rsqrt(jnp.maximum(deg, 1.0))
    dinvb = jnp.broadcast_to(dinv, (BLK, H))
    g = h_ref[...] * dinvb
    ga_ref[...] = g[:, :HH].astype(jnp.bfloat16)
    gb_ref[...] = g[:, HH:].astype(jnp.bfloat16)
    dinv_ref[...] = dinvb


def _tc_b(aa_ref, ab_ref, ga_ref, gb_ref, dinv_ref, b1_ref, w2_ref,
          g2a_ref, g2b_ref):
    dinvb = dinv_ref[...]
    acc = jnp.concatenate([aa_ref[...], ab_ref[...]], axis=1).astype(
        jnp.float32)
    g1 = jnp.concatenate([ga_ref[...], gb_ref[...]], axis=1).astype(
        jnp.float32)
    out1 = jnp.maximum(dinvb * (acc + g1) + b1_ref[...], 0.0)
    g2 = jnp.dot(out1, w2_ref[...], preferred_element_type=jnp.float32) * dinvb
    g2a_ref[...] = g2[:, :HH].astype(jnp.bfloat16)
    g2b_ref[...] = g2[:, HH:].astype(jnp.bfloat16)


def _tc_c(aa_ref, ab_ref, ga_ref, gb_ref, dinv_ref, b2_ref, batch_ref,
          wc_ref, bc_ref, out_ref, psum, pcnt):
    i = pl.program_id(0)

    @pl.when(i == 0)
    def _():
        psum[...] = jnp.zeros_like(psum)
        pcnt[...] = jnp.zeros_like(pcnt)

    dinvb = dinv_ref[...]
    acc = jnp.concatenate([aa_ref[...], ab_ref[...]], axis=1).astype(
        jnp.float32)
    g2 = jnp.concatenate([ga_ref[...], gb_ref[...]], axis=1).astype(
        jnp.float32)
    out2 = jnp.maximum(dinvb * (acc + g2) + b2_ref[...], 0.0)
    ids = batch_ref[0]
    iota = lax.broadcasted_iota(jnp.int32, (G, BLK), 0)
    onehot = (iota == ids).astype(jnp.float32)
    psum[...] += jnp.dot(onehot, out2, preferred_element_type=jnp.float32)
    pcnt[...] += jnp.dot(onehot, jnp.ones((BLK, 8), jnp.float32),
                         preferred_element_type=jnp.float32)

    @pl.when(i == NB - 1)
    def _():
        pooled = psum[...] / jnp.maximum(pcnt[:, 0:1], 1.0)
        out_ref[...] = jnp.dot(
            pooled, wc_ref[...], preferred_element_type=jnp.float32) + bc_ref[...]


def _row_spec(width):
    return pl.BlockSpec((BLK, width), lambda i: (i, 0))


def _full_spec(shape):
    return pl.BlockSpec(shape, lambda i: tuple(0 for _ in shape))


def _half_shapes():
    return [jax.ShapeDtypeStruct((N_PAD, HH), jnp.bfloat16),
            jax.ShapeDtypeStruct((N_PAD, HH), jnp.bfloat16)]


_tc_a1_call = pl.pallas_call(
    _tc_a1,
    grid=(NB,),
    in_specs=[_row_spec(D_IN), _full_spec((D_IN, H))],
    out_specs=_row_spec(H),
    out_shape=jax.ShapeDtypeStruct((N, H), jnp.float32),
)

_tc_a2_call = pl.pallas_call(
    _tc_a2,
    grid=(NB,),
    in_specs=[_row_spec(H), _row_spec(DEG_W), _row_spec(DEG_W)],
    out_specs=[_row_spec(HH), _row_spec(HH), _row_spec(H)],
    out_shape=_half_shapes() + [jax.ShapeDtypeStruct((N, H), jnp.float32)],
)

_tc_b_call = pl.pallas_call(
    _tc_b,
    grid=(NB,),
    in_specs=[_row_spec(HH), _row_spec(HH), _row_spec(HH), _row_spec(HH),
              _row_spec(H), _full_spec((1, H)), _full_spec((H, H))],
    out_specs=[_row_spec(HH), _row_spec(HH)],
    out_shape=_half_shapes(),
)

_tc_c_call = pl.pallas_call(
    _tc_c,
    grid=(NB,),
    in_specs=[_row_spec(HH), _row_spec(HH), _row_spec(HH), _row_spec(HH),
              _row_spec(H), _full_spec((1, H)),
              pl.BlockSpec((1, 1, BLK), lambda i: (i, 0, 0)),
              _full_spec((H, OUT)), _full_spec((1, OUT))],
    out_specs=_full_spec((G, OUT)),
    out_shape=jax.ShapeDtypeStruct((G, OUT), jnp.float32),
    scratch_shapes=[pltpu.VMEM((G, H), jnp.float32),
                    pltpu.VMEM((G, 8), jnp.float32)],
)


@jax.jit
def kernel(x, edge_index, batch, W1, b1, W2, b2, Wc, bc):
    src = edge_index[0]
    dst = edge_index[1]
    pad = E_PAD - E
    src2 = jnp.concatenate([src, jnp.zeros((pad,), jnp.int32)]).reshape(
        TCH, CHUNK)
    dst2 = jnp.concatenate([dst, jnp.full((pad,), N, jnp.int32)]).reshape(
        TCH, CHUNK)
    z16 = jnp.zeros((RPT, DEG_W), jnp.float32)
    z32 = jnp.zeros((RPT, HH), jnp.bfloat16)
    batch3 = batch.reshape(NB, 1, BLK)

    deg0, deg1 = _sc_degree_call()(dst2, z16)
    h1 = _tc_a1_call(x, W1)
    g1a, g1b, dinvb = _tc_a2_call(h1, deg0, deg1)
    a0, a1 = _sc_edge_call()(src2, dst2, g1a, g1b, z32)
    g2a, g2b = _tc_b_call(a0, a1, g1a, g1b, dinvb, b1.reshape(1, H), W2)
    c0, c1 = _sc_edge_call()(src2, dst2, g2a, g2b, z32)
    return _tc_c_call(c0, c1, g2a, g2b, dinvb, b2.reshape(1, H), batch3,
                      Wc, bc.reshape(1, OUT))
